# Initial kernel scaffold; baseline (speedup 1.0000x reference)
import jax
import jax.numpy as jnp
from jax.experimental import pallas as pl


def kernel(input_ids, input_image, params):
    raise NotImplementedError("write your pallas kernel here")


# speedup vs baseline: 1.4374x; 1.4374x over previous
import math
from functools import partial

import jax
import jax.numpy as jnp
from jax.experimental import pallas as pl
from jax.experimental.pallas import tpu as pltpu

HID = 768
NH = 12
HD = HID // NH
FF = 3072
NL = 4
PS = 8
GRID = 8
NPATCH = GRID * GRID * GRID
NTOK = 16
SEQ = NTOK + NPATCH
B = 2
ROWS = B * SEQ
FF_CHUNK = 768


def _ln(x, g, b, eps=1e-5):
    m = jnp.mean(x, axis=-1, keepdims=True)
    xc = x - m
    v = jnp.mean(xc * xc, axis=-1, keepdims=True)
    return xc * jax.lax.rsqrt(v + eps) * g + b


def _mm_t(a, w):
    return jax.lax.dot_general(a, w, (((1,), (1,)), ((), ())),
                               preferred_element_type=jnp.float32)


def _layer_body(x, wq, wk, wv, wo, bo, ln1g, ln1b, ln2g, ln2b, w1, b1, w2, b2):
    h = _ln(x, ln1g, ln1b)
    q = _mm_t(h, wq)
    k = _mm_t(h, wk)
    v = _mm_t(h, wv)
    scale = 1.0 / math.sqrt(HD)
    row = jax.lax.broadcasted_iota(jnp.int32, (SEQ, SEQ), 0)
    col = jax.lax.broadcasted_iota(jnp.int32, (SEQ, SEQ), 1)
    causal = row >= col
    outs = []
    for bi in range(B):
        r0 = bi * SEQ
        head_outs = []
        for hi in range(NH):
            c0 = hi * HD
            qb = q[r0:r0 + SEQ, c0:c0 + HD]
            kb = k[r0:r0 + SEQ, c0:c0 + HD]
            vb = v[r0:r0 + SEQ, c0:c0 + HD]
            logits = jax.lax.dot_general(
                qb, kb, (((1,), (1,)), ((), ())),
                preferred_element_type=jnp.float32) * scale
            logits = jnp.where(causal, logits, jnp.float32(-1e9))
            m = jnp.max(logits, axis=-1, keepdims=True)
            p = jnp.exp(logits - m)
            s = jnp.sum(p, axis=-1, keepdims=True)
            attn = p / s
            head_outs.append(jnp.dot(attn, vb, preferred_element_type=jnp.float32))
        outs.append(jnp.concatenate(head_outs, axis=1))
    o = jnp.concatenate(outs, axis=0)
    x = x + _mm_t(o, wo) + bo
    h2 = _ln(x, ln2g, ln2b)
    acc = x
    for c in range(0, FF, FF_CHUNK):
        w1c = w1[c:c + FF_CHUNK, :]
        b1c = b1[:, c:c + FF_CHUNK]
        w2c = w2[:, c:c + FF_CHUNK]
        ff = jax.nn.gelu(_mm_t(h2, w1c) + b1c)
        acc = acc + jnp.dot(ff, w2c.T, preferred_element_type=jnp.float32)
    return acc + b2


def _layer_kernel(x_ref, wq_ref, wk_ref, wv_ref, wo_ref, bo_ref,
                  ln1g_ref, ln1b_ref, ln2g_ref, ln2b_ref,
                  w1_ref, b1_ref, w2_ref, b2_ref, out_ref):
    args = [r[...] for r in (x_ref, wq_ref, wk_ref, wv_ref, wo_ref, bo_ref,
                             ln1g_ref, ln1b_ref, ln2g_ref, ln2b_ref,
                             w1_ref, b1_ref, w2_ref, b2_ref)]
    out_ref[...] = _layer_body(*args)


def _final_layer_kernel(x_ref, wq_ref, wk_ref, wv_ref, wo_ref, bo_ref,
                        ln1g_ref, ln1b_ref, ln2g_ref, ln2b_ref,
                        w1_ref, b1_ref, w2_ref, b2_ref,
                        nfg_ref, nfb_ref, out_ref):
    args = [r[...] for r in (x_ref, wq_ref, wk_ref, wv_ref, wo_ref, bo_ref,
                             ln1g_ref, ln1b_ref, ln2g_ref, ln2b_ref,
                             w1_ref, b1_ref, w2_ref, b2_ref)]
    y = _layer_body(*args)
    out_ref[...] = _ln(y, nfg_ref[...], nfb_ref[...])


def _embed_layer_kernel(tok_ref, patch_ref, wp_ref, bp_ref,
                        wq_ref, wk_ref, wv_ref, wo_ref, bo_ref,
                        ln1g_ref, ln1b_ref, ln2g_ref, ln2b_ref,
                        w1_ref, b1_ref, w2_ref, b2_ref, out_ref):
    img = jnp.dot(patch_ref[...], wp_ref[...],
                  preferred_element_type=jnp.float32) + bp_ref[...]
    rows = []
    for bi in range(B):
        rows.append(tok_ref[bi * NTOK:(bi + 1) * NTOK, :])
        rows.append(img[bi * NPATCH:(bi + 1) * NPATCH, :])
    x = jnp.concatenate(rows, axis=0)
    args = [r[...] for r in (wq_ref, wk_ref, wv_ref, wo_ref, bo_ref,
                             ln1g_ref, ln1b_ref, ln2g_ref, ln2b_ref,
                             w1_ref, b1_ref, w2_ref, b2_ref)]
    out_ref[...] = _layer_body(x, *args)


_CPARAMS = pltpu.CompilerParams(vmem_limit_bytes=110 * 1024 * 1024)


def _call(kern, args):
    return pl.pallas_call(
        kern,
        out_shape=jax.ShapeDtypeStruct((ROWS, HID), jnp.float32),
        compiler_params=_CPARAMS,
    )(*args)


def _layer_weights(lp):
    r2 = lambda a: a.reshape(1, -1)
    return [lp['Wq'], lp['Wk'], lp['Wv'], lp['Wo'], r2(lp['bo']),
            r2(lp['ln1_g']), r2(lp['ln1_b']), r2(lp['ln2_g']), r2(lp['ln2_b']),
            lp['W1'], r2(lp['b1']), lp['W2'], r2(lp['b2'])]


@jax.jit
def _run(input_ids, input_image, params):
    tok = params['embed'][input_ids].reshape(B * NTOK, HID)
    img = input_image.reshape(B, 1, GRID, PS, GRID, PS, GRID, PS)
    patches = img.transpose(0, 2, 4, 6, 1, 3, 5, 7).reshape(B * NPATCH,
                                                            PS * PS * PS)
    layers = params['layers']
    x = _call(
        _embed_layer_kernel,
        [tok, patches, params['Wp'], params['bp'].reshape(1, HID)]
        + _layer_weights(layers[0]))
    for li in range(1, NL - 1):
        x = _call(_layer_kernel, [x] + _layer_weights(layers[li]))
    x = _call(
        _final_layer_kernel,
        [x] + _layer_weights(layers[NL - 1])
        + [params['nf_g'].reshape(1, HID), params['nf_b'].reshape(1, HID)])
    return x.reshape(B, SEQ, HID)


def kernel(input_ids, input_image, params):
    return _run(input_ids, input_image, params)
